# baseline (speedup 1.0000x reference)
import functools

import jax
import jax.numpy as jnp
from jax import lax
from jax.experimental import pallas as pl
from jax.experimental.pallas import tpu as pltpu
from jax.experimental.pallas import tpu_sc as plsc

N_NODES = 10000
D_OUT = 128
N_EDGES = 320000

NC = 2
NS = 16
NW = NC * NS
K = 128
NBUF = 2

CPW = -(-N_EDGES // (NW * K * NBUF)) * NBUF
NROUNDS = CPW // NBUF
NCHUNKS = CPW * NW
E_PAD = NCHUNKS * K

ACC_ROWS = 10240
SHARD = ACC_ROWS // NS
LAST_ROWS = N_NODES - (NS - 1) * SHARD


def _matmul_body(x_ref, w_ref, o_ref):
    o_ref[...] = jnp.dot(x_ref[...], w_ref[...],
                         preferred_element_type=jnp.float32)


def _matmul(X, W):
    n, d_in = X.shape
    d_out = W.shape[1]
    blk = 2000
    grid = n // blk
    return pl.pallas_call(
        _matmul_body,
        grid=(grid,),
        in_specs=[
            pl.BlockSpec((blk, d_in), lambda i: (i, 0)),
            pl.BlockSpec((d_in, d_out), lambda i: (0, 0)),
        ],
        out_specs=pl.BlockSpec((blk, d_out), lambda i: (i, 0)),
        out_shape=jax.ShapeDtypeStruct((n, d_out), jnp.float32),
    )(X, W)


def _sc_body(wh_hbm, edg_hbm, out_hbm, acc, *scratch):
    ibuf = scratch[0:NBUF]
    rows = scratch[NBUF:2 * NBUF]
    isem = scratch[2 * NBUF:3 * NBUF]
    gsem = scratch[3 * NBUF:4 * NBUF]
    ssem = scratch[4 * NBUF:5 * NBUF]

    cid = lax.axis_index("c")
    sid = lax.axis_index("s")
    wid = sid * NC + cid
    wbase = wid * CPW

    def _zero_row(i, carry):
        for c in range(D_OUT // 16):
            rows[0][i, pl.ds(c * 16, 16)] = jnp.zeros((16,), jnp.float32)
        return carry

    lax.fori_loop(0, K, _zero_row, 0)
    zbase = sid * SHARD
    for j in range(SHARD // K):
        pltpu.sync_copy(rows[0], acc.at[pl.ds(zbase + j * K, K), :])
    plsc.subcore_barrier()

    for b in range(NBUF):
        pltpu.async_copy(edg_hbm.at[wbase + b], ibuf[b], isem[b])
    for b in range(NBUF):
        pltpu.make_async_copy(edg_hbm.at[wbase + b], ibuf[b], isem[b]).wait()
        pltpu.async_copy(wh_hbm.at[ibuf[b].at[1]], rows[b], gsem[b])

    def _round(t, carry):
        for b in range(NBUF):
            pltpu.make_async_copy(
                wh_hbm.at[ibuf[b].at[1]], rows[b], gsem[b]).wait()
            pltpu.async_copy(rows[b], acc.at[ibuf[b].at[0]], ssem[b],
                             add=True)
        for b in range(NBUF):
            m = (t + 1) * NBUF + b
            ch = wbase + jnp.where(m >= CPW, m - CPW, m)
            pltpu.make_async_copy(
                rows[b], acc.at[ibuf[b].at[0]], ssem[b]).wait()
            pltpu.async_copy(edg_hbm.at[ch], ibuf[b], isem[b])
        for b in range(NBUF):
            m = (t + 1) * NBUF + b
            ch = wbase + jnp.where(m >= CPW, m - CPW, m)
            pltpu.make_async_copy(edg_hbm.at[ch], ibuf[b], isem[b]).wait()
            pltpu.async_copy(wh_hbm.at[ibuf[b].at[1]], rows[b], gsem[b])
        return carry

    lax.fori_loop(0, NROUNDS, _round, 0)
    for b in range(NBUF):
        pltpu.make_async_copy(wh_hbm.at[ibuf[b].at[1]], rows[b],
                              gsem[b]).wait()
    plsc.subcore_barrier()

    rb = sid * SHARD

    @pl.when(sid < NS - 1)
    def _():
        pltpu.sync_copy(acc.at[pl.ds(rb, SHARD), :],
                        out_hbm.at[cid, pl.ds(rb, SHARD), :])

    @pl.when(sid == NS - 1)
    def _():
        pltpu.sync_copy(acc.at[pl.ds(rb, LAST_ROWS), :],
                        out_hbm.at[cid, pl.ds(rb, LAST_ROWS), :])


_sc_scatter = functools.partial(
    pl.kernel,
    out_type=jax.ShapeDtypeStruct((NC, N_NODES, D_OUT), jnp.float32),
    mesh=plsc.VectorSubcoreMesh(core_axis_name="c", subcore_axis_name="s"),
    scratch_types=(
        [pltpu.VMEM_SHARED((ACC_ROWS, D_OUT), jnp.float32)]
        + [pltpu.VMEM((2, K), jnp.int32) for _ in range(NBUF)]
        + [pltpu.VMEM((K, D_OUT), jnp.float32) for _ in range(NBUF)]
        + [pltpu.SemaphoreType.DMA for _ in range(3 * NBUF)]
    ),
)(_sc_body)


def _sum_body(p_ref, o_ref):
    o_ref[...] = p_ref[0] + p_ref[1]


def _sum2(parts):
    _, n, d = parts.shape
    blk = 2000
    return pl.pallas_call(
        _sum_body,
        grid=(n // blk,),
        in_specs=[pl.BlockSpec((NC, blk, d), lambda i: (0, i, 0))],
        out_specs=pl.BlockSpec((blk, d), lambda i: (i, 0)),
        out_shape=jax.ShapeDtypeStruct((n, d), jnp.float32),
    )(parts)


def kernel(X, edges, W, a):
    del a
    n = X.shape[0]
    e = edges.shape[1]
    Wh = _matmul(X, W)
    src = edges[0].astype(jnp.int32)
    tgt = edges[1].astype(jnp.int32)
    pad = E_PAD - e
    src = jnp.concatenate([src, jnp.full((pad,), n, jnp.int32)])
    tgt = jnp.concatenate([tgt, jnp.zeros((pad,), jnp.int32)])
    edg = jnp.stack([src.reshape(NCHUNKS, K), tgt.reshape(NCHUNKS, K)],
                    axis=1)
    parts = _sc_scatter(Wh, edg)
    return _sum2(parts)


# speedup vs baseline: 1.4959x; 1.4959x over previous
import functools

import jax
import jax.numpy as jnp
from jax import lax
from jax.experimental import pallas as pl
from jax.experimental.pallas import tpu as pltpu
from jax.experimental.pallas import tpu_sc as plsc

N_NODES = 10000
D_OUT = 128
N_EDGES = 320000

NC = 2
NS = 16
NW = NC * NS
K = 128

CPW = -(-N_EDGES // (NW * K))
NCHUNKS = CPW * NW
E_PAD = NCHUNKS * K

ACC_ROWS = 10240
SHARD = ACC_ROWS // NS
LAST_ROWS = N_NODES - (NS - 1) * SHARD


def _matmul_body(x_ref, w_ref, o_ref):
    o_ref[...] = jnp.dot(x_ref[...], w_ref[...],
                         preferred_element_type=jnp.float32)


def _matmul(X, W):
    n, d_in = X.shape
    d_out = W.shape[1]
    blk = 2000
    grid = n // blk
    return pl.pallas_call(
        _matmul_body,
        grid=(grid,),
        in_specs=[
            pl.BlockSpec((blk, d_in), lambda i: (i, 0)),
            pl.BlockSpec((d_in, d_out), lambda i: (0, 0)),
        ],
        out_specs=pl.BlockSpec((blk, d_out), lambda i: (i, 0)),
        out_shape=jax.ShapeDtypeStruct((n, d_out), jnp.float32),
    )(X, W)


def _sc_body(wh_hbm, edg_hbm, out_hbm, acc, idx_all, rows, gsem):
    cid = lax.axis_index("c")
    sid = lax.axis_index("s")
    wid = sid * NC + cid

    idx_cp = pltpu.async_copy(edg_hbm.at[wid], idx_all, gsem)

    def _zero_row(i, carry):
        for c in range(D_OUT // 16):
            rows[i, pl.ds(c * 16, 16)] = jnp.zeros((16,), jnp.float32)
        return carry

    lax.fori_loop(0, K, _zero_row, 0)
    zbase = sid * SHARD
    for j in range(SHARD // K):
        pltpu.sync_copy(rows, acc.at[pl.ds(zbase + j * K, K), :])
    plsc.subcore_barrier()

    idx_cp.wait()

    def _chunk(c, carry):
        pltpu.async_copy(wh_hbm.at[idx_all.at[c, 1]], rows, gsem).wait()
        pltpu.sync_copy(rows, acc.at[idx_all.at[c, 0]], add=True)
        return carry

    lax.fori_loop(0, CPW, _chunk, 0)
    plsc.subcore_barrier()

    rb = sid * SHARD

    @pl.when(sid < NS - 1)
    def _():
        pltpu.sync_copy(acc.at[pl.ds(rb, SHARD), :],
                        out_hbm.at[cid, pl.ds(rb, SHARD), :])

    @pl.when(sid == NS - 1)
    def _():
        pltpu.sync_copy(acc.at[pl.ds(rb, LAST_ROWS), :],
                        out_hbm.at[cid, pl.ds(rb, LAST_ROWS), :])


_sc_scatter = functools.partial(
    pl.kernel,
    out_type=jax.ShapeDtypeStruct((NC, N_NODES, D_OUT), jnp.float32),
    mesh=plsc.VectorSubcoreMesh(core_axis_name="c", subcore_axis_name="s"),
    scratch_types=[
        pltpu.VMEM_SHARED((ACC_ROWS, D_OUT), jnp.float32),
        pltpu.VMEM((CPW, 2, K), jnp.int32),
        pltpu.VMEM((K, D_OUT), jnp.float32),
        pltpu.SemaphoreType.DMA,
    ],
)(_sc_body)


def _sum_body(p_ref, o_ref):
    o_ref[...] = p_ref[0] + p_ref[1]


def _sum2(parts):
    _, n, d = parts.shape
    blk = 2000
    return pl.pallas_call(
        _sum_body,
        grid=(n // blk,),
        in_specs=[pl.BlockSpec((NC, blk, d), lambda i: (0, i, 0))],
        out_specs=pl.BlockSpec((blk, d), lambda i: (i, 0)),
        out_shape=jax.ShapeDtypeStruct((n, d), jnp.float32),
    )(parts)


def kernel(X, edges, W, a):
    del a
    n = X.shape[0]
    e = edges.shape[1]
    Wh = _matmul(X, W)
    src = edges[0].astype(jnp.int32)
    tgt = edges[1].astype(jnp.int32)
    pad = E_PAD - e
    src = jnp.concatenate([src, jnp.full((pad,), n, jnp.int32)])
    tgt = jnp.concatenate([tgt, jnp.zeros((pad,), jnp.int32)])
    edg = jnp.stack([src.reshape(NW, CPW, K),
                     tgt.reshape(NW, CPW, K)], axis=2)
    parts = _sc_scatter(Wh, edg)
    return _sum2(parts)
